# Initial kernel scaffold; baseline (speedup 1.0000x reference)
import jax
import jax.numpy as jnp
from jax.experimental import pallas as pl


def kernel(x, edge_index, W1, b1, W2, b2):
    raise NotImplementedError("write your pallas kernel here")


# speedup vs baseline: 14.5137x; 14.5137x over previous
import functools

import jax
import jax.numpy as jnp
from jax import lax
from jax.experimental import pallas as pl
from jax.experimental.pallas import tpu as pltpu
from jax.experimental.pallas import tpu_sc as plsc

_N, _E, _D, _H, _C = 10000, 320000, 128, 128, 64
_NC, _NS = 2, 16
_NW = _NC * _NS
_UNIT = 128
_NUNITS = _E // _UNIT
_BASE = _NUNITS // _NW
_EXTRA = _NUNITS - _BASE * _NW
_NP = 10240
_RPS = _NP // _NS
_DEGW = 128

_mesh = lambda: plsc.VectorSubcoreMesh(core_axis_name="c", subcore_axis_name="s")


def _zero16():
    return jnp.zeros((16,), jnp.float32)


def _zero_acc_slice(zrows, acc, base):
    for t in range(_RPS // _UNIT):
        pltpu.sync_copy(zrows, acc.at[pl.ds(base + t * _UNIT, _UNIT)])


def _make_deg():
    @functools.partial(
        pl.kernel,
        out_type=jax.ShapeDtypeStruct((_NC, _NP, _DEGW), jnp.float32),
        mesh=_mesh(),
        scratch_types=[
            pltpu.VMEM((1, _UNIT), jnp.int32),
            pltpu.VMEM((_UNIT, _DEGW), jnp.float32),
            pltpu.VMEM((_UNIT, _DEGW), jnp.float32),
            pltpu.VMEM_SHARED((_NP, _DEGW), jnp.float32),
        ],
    )
    def deg_kernel(dsts, out, dstv, onesv, zv, acc):
        cid = lax.axis_index("c")
        sid = lax.axis_index("s")
        wid = sid * _NC + cid
        base = sid * _RPS

        ones16 = jnp.ones((16,), jnp.float32)
        z16 = _zero16()

        def fill(r, carry):
            for cidx in range(_DEGW // 16):
                onesv[r, pl.ds(cidx * 16, 16)] = ones16
                zv[r, pl.ds(cidx * 16, 16)] = z16
            return carry

        lax.fori_loop(0, _UNIT, fill, 0)
        _zero_acc_slice(zv, acc, base)
        plsc.subcore_barrier()

        def body(u):
            off = u * _UNIT
            pltpu.sync_copy(dsts.at[pl.ds(off, _UNIT)], dstv.at[0])
            pltpu.sync_copy(onesv, acc.at[dstv.at[0]], add=True)

        def loop(i, carry):
            body(wid + i * _NW)
            return carry

        lax.fori_loop(0, _BASE, loop, 0)

        @pl.when(wid < _EXTRA)
        def _():
            body(wid + _BASE * _NW)

        plsc.subcore_barrier()
        pltpu.sync_copy(acc.at[pl.ds(base, _RPS)],
                        out.at[cid, pl.ds(base, _RPS)])

    return deg_kernel


def _make_agg(F):
    @functools.partial(
        pl.kernel,
        out_type=jax.ShapeDtypeStruct((_NC, _NP, F), jnp.float32),
        mesh=_mesh(),
        scratch_types=[
            pltpu.VMEM((1, _UNIT), jnp.int32),
            pltpu.VMEM((1, _UNIT), jnp.int32),
            pltpu.VMEM((_UNIT, F), jnp.float32),
            pltpu.VMEM_SHARED((_NP, F), jnp.float32),
        ],
    )
    def agg_kernel(srcs, dsts, h, out, srcv, dstv, rows, acc):
        cid = lax.axis_index("c")
        sid = lax.axis_index("s")
        wid = sid * _NC + cid
        base = sid * _RPS

        z16 = _zero16()

        def zfill(r, carry):
            for cidx in range(F // 16):
                rows[r, pl.ds(cidx * 16, 16)] = z16
            return carry

        lax.fori_loop(0, _UNIT, zfill, 0)
        _zero_acc_slice(rows, acc, base)
        plsc.subcore_barrier()

        def body(u):
            off = u * _UNIT
            pltpu.sync_copy(srcs.at[pl.ds(off, _UNIT)], srcv.at[0])
            pltpu.sync_copy(dsts.at[pl.ds(off, _UNIT)], dstv.at[0])
            pltpu.sync_copy(h.at[srcv.at[0]], rows)
            pltpu.sync_copy(rows, acc.at[dstv.at[0]], add=True)

        def loop(i, carry):
            body(wid + i * _NW)
            return carry

        lax.fori_loop(0, _BASE, loop, 0)

        @pl.when(wid < _EXTRA)
        def _():
            body(wid + _BASE * _NW)

        plsc.subcore_barrier()
        pltpu.sync_copy(acc.at[pl.ds(base, _RPS)],
                        out.at[cid, pl.ds(base, _RPS)])

    return agg_kernel


_deg_call = _make_deg()
_agg_h = _make_agg(_H)

_BLK = 1000
_GRID = _N // _BLK


def _dinv_of(d_ref):
    s = d_ref[0, :, 0:1] + d_ref[1, :, 0:1] + 1.0
    return lax.rsqrt(s)


def _mm1(x, W1, deg):
    def body(x_ref, w_ref, d_ref, o_ref):
        dinv = _dinv_of(d_ref)
        h = jnp.dot(x_ref[...], w_ref[...], preferred_element_type=jnp.float32)
        o_ref[...] = h * dinv

    return pl.pallas_call(
        body,
        grid=(_GRID,),
        in_specs=[
            pl.BlockSpec((_BLK, _D), lambda i: (i, 0)),
            pl.BlockSpec((_D, _H), lambda i: (0, 0)),
            pl.BlockSpec((_NC, _BLK, _DEGW), lambda i: (0, i, 0)),
        ],
        out_specs=pl.BlockSpec((_BLK, _H), lambda i: (i, 0)),
        out_shape=jax.ShapeDtypeStruct((_N, _H), jnp.float32),
    )(x, W1, deg)


def _mm2(acc1, h1s, deg, x, b1, W2):
    def body(a_ref, h_ref, d_ref, x_ref, b_ref, w_ref, o_ref):
        dinv = _dinv_of(d_ref)
        agg = a_ref[0] + a_ref[1] + h_ref[...]
        out1 = agg * dinv + b_ref[...]
        z = jnp.maximum(out1, 0.0) + x_ref[...]
        h2 = jnp.dot(z, w_ref[...], preferred_element_type=jnp.float32)
        o_ref[...] = jnp.concatenate([h2 * dinv, jnp.zeros_like(h2)], axis=1)

    return pl.pallas_call(
        body,
        grid=(_GRID,),
        in_specs=[
            pl.BlockSpec((_NC, _BLK, _H), lambda i: (0, i, 0)),
            pl.BlockSpec((_BLK, _H), lambda i: (i, 0)),
            pl.BlockSpec((_NC, _BLK, _DEGW), lambda i: (0, i, 0)),
            pl.BlockSpec((_BLK, _D), lambda i: (i, 0)),
            pl.BlockSpec((_H,), lambda i: (0,)),
            pl.BlockSpec((_H, _C), lambda i: (0, 0)),
        ],
        out_specs=pl.BlockSpec((_BLK, 2 * _C), lambda i: (i, 0)),
        out_shape=jax.ShapeDtypeStruct((_N, 2 * _C), jnp.float32),
    )(acc1, h1s, deg, x, b1, W2)


def _mm3(acc2, h2s, deg, b2):
    def body(a_ref, h_ref, d_ref, b_ref, o_ref):
        dinv = _dinv_of(d_ref)
        agg = a_ref[0, :, : _C] + a_ref[1, :, : _C] + h_ref[:, : _C]
        o_ref[...] = agg * dinv + b_ref[...]

    return pl.pallas_call(
        body,
        grid=(_GRID,),
        in_specs=[
            pl.BlockSpec((_NC, _BLK, 2 * _C), lambda i: (0, i, 0)),
            pl.BlockSpec((_BLK, 2 * _C), lambda i: (i, 0)),
            pl.BlockSpec((_NC, _BLK, _DEGW), lambda i: (0, i, 0)),
            pl.BlockSpec((_C,), lambda i: (0,)),
        ],
        out_specs=pl.BlockSpec((_BLK, _C), lambda i: (i, 0)),
        out_shape=jax.ShapeDtypeStruct((_N, _C), jnp.float32),
    )(acc2, h2s, deg, b2)


def kernel(x, edge_index, W1, b1, W2, b2):
    ei = edge_index.astype(jnp.int32)
    srcs = ei[0]
    dsts = ei[1]
    deg = _deg_call(dsts)
    h1s = _mm1(x, W1, deg)
    acc1 = _agg_h(srcs, dsts, h1s)
    h2s = _mm2(acc1, h1s, deg, x, b1, W2)
    acc2 = _agg_h(srcs, dsts, h2s)
    y = _mm3(acc2, h2s, deg, b2)
    return y
